# Initial kernel scaffold; baseline (speedup 1.0000x reference)
import functools

import jax
import jax.numpy as jnp
from jax import lax
from jax.experimental import pallas as pl
from jax.experimental.pallas import tpu as pltpu
from jax.experimental.pallas import tpu_sc as plsc

N_USERS = 50000
N_ITEMS = 50000
NTOT = N_USERS + N_ITEMS
D = 32
K_LAYERS = 3

NC, NS, L = 2, 16, 16
HALF = NTOT // NC
ACC_ROWS = 50016
TRASH = HALF
ZSTRIPE = ACC_ROWS // NS
WSTRIPE = HALF // NS

K = 8
CHUNK = K * 128
N_EDGES = 1600000
NCHUNK = -(-N_EDGES // (NS * CHUNK))
EPT = NCHUNK * CHUNK
E_PAD = NS * EPT


def _layer_body(e_hbm, src_hbm, dst_hbm, w_hbm, out_hbm,
                acc_sh, src_v, dst_v, lid_v, w_flat, msg_v, gsem):
    c = lax.axis_index("c")
    s = lax.axis_index("s")
    base = c * HALF

    @pl.loop(0, CHUNK)
    def _(i):
        z = jnp.zeros((L,), jnp.float32)
        msg_v[i, pl.ds(0, L)] = z
        msg_v[i, pl.ds(L, L)] = z

    z0 = s * ZSTRIPE
    pltpu.sync_copy(msg_v.at[pl.ds(0, 1024)], acc_sh.at[pl.ds(z0, 1024)])
    pltpu.sync_copy(msg_v.at[pl.ds(0, 1024)], acc_sh.at[pl.ds(z0 + 1024, 1024)])
    pltpu.sync_copy(msg_v.at[pl.ds(0, 1024)], acc_sh.at[pl.ds(z0 + 2048, 1024)])
    pltpu.sync_copy(msg_v.at[pl.ds(0, ZSTRIPE - 3072)],
                    acc_sh.at[pl.ds(z0 + 3072, ZSTRIPE - 3072)])
    plsc.subcore_barrier()

    @pl.loop(0, NCHUNK)
    def _(ch):
        row0 = s * (EPT // 128) + ch * K
        flat0 = s * EPT + ch * CHUNK
        pltpu.sync_copy(src_hbm.at[pl.ds(row0, K)], src_v)
        pltpu.sync_copy(dst_hbm.at[pl.ds(row0, K)], dst_v)
        pltpu.sync_copy(w_hbm.at[pl.ds(flat0, CHUNK)], w_flat)

        descs = [
            pltpu.async_copy(e_hbm.at[src_v.at[j]],
                             msg_v.at[pl.ds(j * 128, 128)], gsem)
            for j in range(K)
        ]

        for j in range(K):
            @pl.loop(0, 128 // L)
            def _(g, j=j):
                dv = dst_v[j, pl.ds(g * L, L)]
                t = dv - base
                ok = (dv >= base) & (dv < base + HALF)
                lid_v[j, pl.ds(g * L, L)] = jnp.where(ok, t, TRASH)

        for d_ in descs:
            d_.wait()

        @pl.loop(0, CHUNK)
        def _(i):
            wv = w_flat[i]
            msg_v[i, pl.ds(0, L)] = msg_v[i, pl.ds(0, L)] * wv
            msg_v[i, pl.ds(L, L)] = msg_v[i, pl.ds(L, L)] * wv

        for j in range(K):
            pltpu.sync_copy(msg_v.at[pl.ds(j * 128, 128)],
                            acc_sh.at[lid_v.at[j]], add=True)

    plsc.subcore_barrier()

    w0 = s * WSTRIPE
    pltpu.sync_copy(acc_sh.at[pl.ds(w0, WSTRIPE)],
                    out_hbm.at[pl.ds(base + w0, WSTRIPE)])


def _make_layer():
    mesh = plsc.VectorSubcoreMesh(core_axis_name="c", subcore_axis_name="s",
                                  num_cores=NC, num_subcores=NS)
    return pl.kernel(
        _layer_body,
        out_type=jax.ShapeDtypeStruct((NTOT, D), jnp.float32),
        mesh=mesh,
        scratch_types=[
            pltpu.VMEM_SHARED((ACC_ROWS, D), jnp.float32),
            pltpu.VMEM((K, 128), jnp.int32),
            pltpu.VMEM((K, 128), jnp.int32),
            pltpu.VMEM((K, 128), jnp.int32),
            pltpu.VMEM((CHUNK,), jnp.float32),
            pltpu.VMEM((CHUNK, D), jnp.float32),
            pltpu.SemaphoreType.DMA,
        ],
    )


def _mean_body(a_ref, b_ref, c_ref, d_ref, o_ref):
    o_ref[...] = (a_ref[...] + b_ref[...] + c_ref[...] + d_ref[...]) * 0.25


def kernel(user_emb, item_emb, edge_index, edge_weight):
    e0 = jnp.concatenate([user_emb, item_emb], axis=0)
    pad = E_PAD - N_EDGES
    src = jnp.pad(edge_index[0], (0, pad)).reshape(E_PAD // 128, 128)
    dst = jnp.pad(edge_index[1], (0, pad)).reshape(E_PAD // 128, 128)
    w = jnp.pad(edge_weight, (0, pad))

    layer = _make_layer()
    e1 = layer(e0, src, dst, w)
    e2 = layer(e1, src, dst, w)
    e3 = layer(e2, src, dst, w)

    blk = 1000
    z = pl.pallas_call(
        _mean_body,
        out_shape=jax.ShapeDtypeStruct((NTOT, D), jnp.float32),
        grid=(NTOT // blk,),
        in_specs=[pl.BlockSpec((blk, D), lambda i: (i, 0))] * 4,
        out_specs=pl.BlockSpec((blk, D), lambda i: (i, 0)),
    )(e0, e1, e2, e3)

    return z[:N_USERS], z[N_USERS:]


# speedup vs baseline: 7.2867x; 7.2867x over previous
import functools

import jax
import jax.numpy as jnp
from jax import lax
from jax.experimental import pallas as pl
from jax.experimental.pallas import tpu as pltpu
from jax.experimental.pallas import tpu_sc as plsc

N_USERS = 50000
N_ITEMS = 50000
NTOT = N_USERS + N_ITEMS
D = 32
K_LAYERS = 3

NC, NS, L = 2, 16, 16
HALF = NTOT // NC
ACC_ROWS = 50048
TRASH = HALF
ZSTRIPE = ACC_ROWS // NS
WSTRIPE = ZSTRIPE
WLAST = HALF - (NS - 1) * WSTRIPE

K = 4
CHUNK = K * 128
N_EDGES = 1600000
NCHUNK = -(-N_EDGES // (NS * CHUNK))
EPT = NCHUNK * CHUNK
E_PAD = NS * EPT


def _layer_body(e_hbm, src_hbm, dst_hbm, w_hbm, out_hbm,
                acc_sh, src_v, dst_v, lid_v, w_flat, msg_v, gsem):
    c = lax.axis_index("c")
    s = lax.axis_index("s")
    base = c * HALF

    @pl.loop(0, CHUNK)
    def _(i):
        z = jnp.zeros((L,), jnp.float32)
        msg_v[i, pl.ds(0, L)] = z
        msg_v[i, pl.ds(L, L)] = z

    z0 = s * ZSTRIPE
    nz = ZSTRIPE // CHUNK

    @pl.loop(0, nz)
    def _(i):
        pltpu.sync_copy(msg_v.at[pl.ds(0, CHUNK)],
                        acc_sh.at[pl.ds(z0 + i * CHUNK, CHUNK)])

    pltpu.sync_copy(msg_v.at[pl.ds(0, ZSTRIPE - nz * CHUNK)],
                    acc_sh.at[pl.ds(z0 + nz * CHUNK, ZSTRIPE - nz * CHUNK)])
    plsc.subcore_barrier()

    @pl.loop(0, NCHUNK)
    def _(ch):
        row0 = s * (EPT // 128) + ch * K
        flat0 = s * EPT + ch * CHUNK
        pltpu.sync_copy(src_hbm.at[pl.ds(row0, K)], src_v)
        pltpu.sync_copy(dst_hbm.at[pl.ds(row0, K)], dst_v)
        pltpu.sync_copy(w_hbm.at[pl.ds(flat0, CHUNK)], w_flat)

        descs = [
            pltpu.async_copy(e_hbm.at[src_v.at[j]],
                             msg_v.at[pl.ds(j * 128, 128)], gsem)
            for j in range(K)
        ]

        for j in range(K):
            @pl.loop(0, 128 // L)
            def _(g, j=j):
                dv = dst_v[j, pl.ds(g * L, L)]
                t = dv - base
                ok = (dv >= base) & (dv < base + HALF)
                lid_v[j, pl.ds(g * L, L)] = jnp.where(ok, t, TRASH)

        for d_ in descs:
            d_.wait()

        @pl.loop(0, CHUNK // L)
        def _(g):
            w16 = w_flat[pl.ds(g * L, L)]
            for r in range(L):
                i = g * L + r
                wv = w16[r]
                msg_v[i, pl.ds(0, L)] = msg_v[i, pl.ds(0, L)] * wv
                msg_v[i, pl.ds(L, L)] = msg_v[i, pl.ds(L, L)] * wv

        for j in range(K):
            pltpu.sync_copy(msg_v.at[pl.ds(j * 128, 128)],
                            acc_sh.at[lid_v.at[j]], add=True)

    plsc.subcore_barrier()

    w0 = s * WSTRIPE

    @pl.when(s < NS - 1)
    def _():
        pltpu.sync_copy(acc_sh.at[pl.ds(w0, WSTRIPE)],
                        out_hbm.at[pl.ds(base + w0, WSTRIPE)])

    @pl.when(s == NS - 1)
    def _():
        pltpu.sync_copy(acc_sh.at[pl.ds(w0, WLAST)],
                        out_hbm.at[pl.ds(base + w0, WLAST)])


def _make_layer():
    mesh = plsc.VectorSubcoreMesh(core_axis_name="c", subcore_axis_name="s",
                                  num_cores=NC, num_subcores=NS)
    return pl.kernel(
        _layer_body,
        out_type=jax.ShapeDtypeStruct((NTOT, D), jnp.float32),
        mesh=mesh,
        scratch_types=[
            pltpu.VMEM_SHARED((ACC_ROWS, D), jnp.float32),
            pltpu.VMEM((K, 128), jnp.int32),
            pltpu.VMEM((K, 128), jnp.int32),
            pltpu.VMEM((K, 128), jnp.int32),
            pltpu.VMEM((CHUNK,), jnp.float32),
            pltpu.VMEM((CHUNK, D), jnp.float32),
            pltpu.SemaphoreType.DMA,
        ],
        compiler_params=pltpu.CompilerParams(use_tc_tiling_on_sc=False),
    )


def _mean_body(a_ref, b_ref, c_ref, d_ref, o_ref):
    o_ref[...] = (a_ref[...] + b_ref[...] + c_ref[...] + d_ref[...]) * 0.25


def kernel(user_emb, item_emb, edge_index, edge_weight):
    e0 = jnp.concatenate([user_emb, item_emb], axis=0)
    pad = E_PAD - N_EDGES
    src = jnp.pad(edge_index[0], (0, pad)).reshape(E_PAD // 128, 128)
    dst = jnp.pad(edge_index[1], (0, pad)).reshape(E_PAD // 128, 128)
    w = jnp.pad(edge_weight, (0, pad))

    layer = _make_layer()
    e1 = layer(e0, src, dst, w)
    e2 = layer(e1, src, dst, w)
    e3 = layer(e2, src, dst, w)

    blk = 1000
    z = pl.pallas_call(
        _mean_body,
        out_shape=jax.ShapeDtypeStruct((NTOT, D), jnp.float32),
        grid=(NTOT // blk,),
        in_specs=[pl.BlockSpec((blk, D), lambda i: (i, 0))] * 4,
        out_specs=pl.BlockSpec((blk, D), lambda i: (i, 0)),
    )(e0, e1, e2, e3)

    return z[:N_USERS], z[N_USERS:]


# speedup vs baseline: 12.1720x; 1.6704x over previous
import jax
import jax.numpy as jnp
from jax import lax
from jax.experimental import pallas as pl
from jax.experimental.pallas import tpu as pltpu
from jax.experimental.pallas import tpu_sc as plsc

N_USERS = 50000
N_ITEMS = 50000
NR = N_USERS + N_ITEMS
D = 32
DH = D // 2

NC, NS, L = 2, 16, 16
NRP = 100096
ZS = NRP // NS
WLAST = NR - (NS - 1) * ZS

K = 8
CHUNK = K * 128
N_EDGES = 1600000
NCHUNK = -(-N_EDGES // (NS * CHUNK))
EPT = NCHUNK * CHUNK
E_PAD = NS * EPT


def _sc_body(e0_hbm, src_hbm, dst_hbm, w_hbm, o1_hbm, o2_hbm, o3_hbm,
             acc_sh, src_v, dst_v, w_v, msg_v, isem, gsem, ssem):
    c = lax.axis_index("c")
    s = lax.axis_index("s")

    def one_layer(in3, out3):
        in_ref = in3.at[c]
        out_ref = out3.at[c]

        @pl.loop(0, CHUNK)
        def _(i):
            msg_v[i, pl.ds(0, L)] = jnp.zeros((L,), jnp.float32)

        z0 = s * ZS

        @pl.loop(0, ZS // CHUNK)
        def _(i):
            pltpu.sync_copy(msg_v.at[pl.ds(0, CHUNK)],
                            acc_sh.at[pl.ds(z0 + i * CHUNK, CHUNK)])

        pltpu.sync_copy(msg_v.at[pl.ds(0, ZS % CHUNK)],
                        acc_sh.at[pl.ds(z0 + (ZS // CHUNK) * CHUNK,
                                        ZS % CHUNK)])
        plsc.subcore_barrier()

        @pl.loop(0, NCHUNK)
        def _(ch):
            row0 = s * (EPT // 128) + ch * K
            flat0 = s * EPT + ch * CHUNK
            di = [pltpu.async_copy(src_hbm.at[pl.ds(row0, K)], src_v, isem),
                  pltpu.async_copy(dst_hbm.at[pl.ds(row0, K)], dst_v, isem),
                  pltpu.async_copy(w_hbm.at[pl.ds(flat0, CHUNK)], w_v, isem)]
            for d_ in di:
                d_.wait()

            gs = [pltpu.async_copy(in_ref.at[src_v.at[j]],
                                   msg_v.at[pl.ds(j * 128, 128)], gsem)
                  for j in range(K)]
            for d_ in gs:
                d_.wait()

            @pl.loop(0, CHUNK // L)
            def _(g):
                w16 = w_v[pl.ds(g * L, L)]
                for r in range(L):
                    i = g * L + r
                    msg_v[i, pl.ds(0, L)] = msg_v[i, pl.ds(0, L)] * w16[r]

            ss = [pltpu.async_copy(msg_v.at[pl.ds(j * 128, 128)],
                                   acc_sh.at[dst_v.at[j]], ssem, add=True)
                  for j in range(K)]
            for d_ in ss:
                d_.wait()

        plsc.subcore_barrier()

        w0 = s * ZS

        @pl.when(s < NS - 1)
        def _():
            pltpu.sync_copy(acc_sh.at[pl.ds(w0, ZS)],
                            out_ref.at[pl.ds(w0, ZS)])

        @pl.when(s == NS - 1)
        def _():
            pltpu.sync_copy(acc_sh.at[pl.ds(w0, WLAST)],
                            out_ref.at[pl.ds(w0, WLAST)])

        plsc.subcore_barrier()

    one_layer(e0_hbm, o1_hbm)
    one_layer(o1_hbm, o2_hbm)
    one_layer(o2_hbm, o3_hbm)


def _make_sc_kernel():
    mesh = plsc.VectorSubcoreMesh(core_axis_name="c", subcore_axis_name="s",
                                  num_cores=NC, num_subcores=NS)
    half = jax.ShapeDtypeStruct((NC, NR, DH), jnp.float32)
    return pl.kernel(
        _sc_body,
        out_type=(half, half, half),
        mesh=mesh,
        scratch_types=[
            pltpu.VMEM_SHARED((NRP, DH), jnp.float32),
            pltpu.VMEM((K, 128), jnp.int32),
            pltpu.VMEM((K, 128), jnp.int32),
            pltpu.VMEM((CHUNK,), jnp.float32),
            pltpu.VMEM((CHUNK, DH), jnp.float32),
            pltpu.SemaphoreType.DMA,
            pltpu.SemaphoreType.DMA,
            pltpu.SemaphoreType.DMA,
        ],
        compiler_params=pltpu.CompilerParams(use_tc_tiling_on_sc=False),
    )


def _mean_body(a_ref, b_ref, c_ref, d_ref, o_ref):
    o_ref[0] = (a_ref[0] + b_ref[0] + c_ref[0] + d_ref[0]) * 0.25


def kernel(user_emb, item_emb, edge_index, edge_weight):
    e0 = jnp.concatenate([user_emb, item_emb], axis=0)
    e0_st = jnp.stack([e0[:, :DH], e0[:, DH:]], axis=0)
    pad = E_PAD - N_EDGES
    src = jnp.pad(edge_index[0], (0, pad)).reshape(E_PAD // 128, 128)
    dst = jnp.pad(edge_index[1], (0, pad)).reshape(E_PAD // 128, 128)
    w = jnp.pad(edge_weight, (0, pad))

    o1, o2, o3 = _make_sc_kernel()(e0_st, src, dst, w)

    blk = 800
    zs = pl.pallas_call(
        _mean_body,
        out_shape=jax.ShapeDtypeStruct((NC, NR, DH), jnp.float32),
        grid=(NC, NR // blk),
        in_specs=[pl.BlockSpec((1, blk, DH), lambda i, j: (i, j, 0))] * 4,
        out_specs=pl.BlockSpec((1, blk, DH), lambda i, j: (i, j, 0)),
    )(e0_st, o1, o2, o3)

    z = jnp.concatenate([zs[0], zs[1]], axis=1)
    return z[:N_USERS], z[N_USERS:]


# speedup vs baseline: 15.5024x; 1.2736x over previous
import jax
import jax.numpy as jnp
from jax import lax
from jax.experimental import pallas as pl
from jax.experimental.pallas import tpu as pltpu
from jax.experimental.pallas import tpu_sc as plsc

N_USERS = 50000
N_ITEMS = 50000
NR = N_USERS + N_ITEMS
D = 32
DH = D // 2

NC, NS, L = 2, 16, 16
NRP = 100096
ZS = NRP // NS
WLAST = NR - (NS - 1) * ZS

K = 4
CHUNK = K * 128
N_EDGES = 1600000
NCHUNK = -(-N_EDGES // (NS * CHUNK))
EPT = NCHUNK * CHUNK
E_PAD = NS * EPT
NQ = 4


def _sc_body(e0_hbm, src_hbm, dst_hbm, w_hbm, o1_hbm, o2_hbm, o3_hbm,
             acc_sh, src_v, dst_v, w_v, msg_v,
             isem0, isem1, isem2, isem3, gsem0, gsem1, ssem0, ssem1):
    c = lax.axis_index("c")
    s = lax.axis_index("s")
    isems = (isem0, isem1, isem2, isem3)
    gsems = (gsem0, gsem1)
    ssems = (ssem0, ssem1)

    def one_layer(in3, out3):
        in_ref = in3.at[c]
        out_ref = out3.at[c]

        @pl.loop(0, CHUNK)
        def _(i):
            msg_v[0, i, pl.ds(0, L)] = jnp.zeros((L,), jnp.float32)

        z0 = s * ZS

        @pl.loop(0, ZS // CHUNK)
        def _(i):
            pltpu.sync_copy(msg_v.at[0].at[pl.ds(0, CHUNK)],
                            acc_sh.at[pl.ds(z0 + i * CHUNK, CHUNK)])

        pltpu.sync_copy(msg_v.at[0].at[pl.ds(0, ZS % CHUNK)],
                        acc_sh.at[pl.ds(z0 + (ZS // CHUNK) * CHUNK,
                                        ZS % CHUNK)])
        plsc.subcore_barrier()

        def idx_issue(ch, q):
            row0 = s * (EPT // 128) + ch * K
            flat0 = s * EPT + ch * CHUNK
            pltpu.async_copy(src_hbm.at[pl.ds(row0, K)], src_v.at[q],
                             isems[q])
            pltpu.async_copy(dst_hbm.at[pl.ds(row0, K)], dst_v.at[q],
                             isems[q])
            pltpu.async_copy(w_hbm.at[pl.ds(flat0, CHUNK)], w_v.at[q],
                             isems[q])

        def idx_wait(ch, q):
            row0 = s * (EPT // 128) + ch * K
            flat0 = s * EPT + ch * CHUNK
            pltpu.make_async_copy(src_hbm.at[pl.ds(row0, K)], src_v.at[q],
                                  isems[q]).wait()
            pltpu.make_async_copy(dst_hbm.at[pl.ds(row0, K)], dst_v.at[q],
                                  isems[q]).wait()
            pltpu.make_async_copy(w_hbm.at[pl.ds(flat0, CHUNK)], w_v.at[q],
                                  isems[q]).wait()

        def gather_issue(b, q):
            for j in range(K):
                pltpu.async_copy(in_ref.at[src_v.at[q].at[j]],
                                 msg_v.at[b].at[pl.ds(j * 128, 128)],
                                 gsems[b])

        def gather_wait(b, q):
            for j in range(K):
                pltpu.make_async_copy(in_ref.at[src_v.at[q].at[j]],
                                      msg_v.at[b].at[pl.ds(j * 128, 128)],
                                      gsems[b]).wait()

        def multiply(b, q):
            @pl.loop(0, CHUNK // L)
            def _(g):
                w16 = w_v[q, pl.ds(g * L, L)]
                for r in range(L):
                    i = g * L + r
                    msg_v[b, i, pl.ds(0, L)] = msg_v[b, i, pl.ds(0, L)] * w16[r]

        def scatter_issue(b, q):
            for j in range(K):
                pltpu.async_copy(msg_v.at[b].at[pl.ds(j * 128, 128)],
                                 acc_sh.at[dst_v.at[q].at[j]],
                                 ssems[b], add=True)

        def scatter_wait(b, q):
            for j in range(K):
                pltpu.make_async_copy(msg_v.at[b].at[pl.ds(j * 128, 128)],
                                      acc_sh.at[dst_v.at[q].at[j]],
                                      ssems[b]).wait()

        idx_issue(0, 0)
        idx_issue(1, 1)
        idx_issue(2, 2)
        idx_wait(0, 0)
        gather_issue(0, 0)

        QN = NCHUNK // NQ

        @pl.loop(0, QN)
        def _(it):
            for sub in range(NQ):
                ch = it * NQ + sub
                b, bp = sub % 2, 1 - sub % 2
                q, qn, qp = sub, (sub + 1) % NQ, (sub - 1) % NQ

                gather_wait(b, q)

                if sub == 0:
                    @pl.when(it > 0)
                    def _():
                        scatter_wait(bp, qp)
                else:
                    scatter_wait(bp, qp)

                if sub < NQ - 1:
                    idx_wait(ch + 1, qn)
                    gather_issue(bp, qn)
                else:
                    @pl.when(it < QN - 1)
                    def _():
                        idx_wait(ch + 1, qn)
                        gather_issue(bp, qn)

                multiply(b, q)
                scatter_issue(b, q)

                if sub == 0:
                    idx_issue(ch + 3, (sub + 3) % NQ)
                else:
                    @pl.when(it < QN - 1)
                    def _():
                        idx_issue(ch + 3, (sub + 3) % NQ)

        scatter_wait(1, 3)
        plsc.subcore_barrier()

        w0 = s * ZS

        @pl.when(s < NS - 1)
        def _():
            pltpu.sync_copy(acc_sh.at[pl.ds(w0, ZS)],
                            out_ref.at[pl.ds(w0, ZS)])

        @pl.when(s == NS - 1)
        def _():
            pltpu.sync_copy(acc_sh.at[pl.ds(w0, WLAST)],
                            out_ref.at[pl.ds(w0, WLAST)])

        plsc.subcore_barrier()

    one_layer(e0_hbm, o1_hbm)
    one_layer(o1_hbm, o2_hbm)
    one_layer(o2_hbm, o3_hbm)


def _make_sc_kernel():
    mesh = plsc.VectorSubcoreMesh(core_axis_name="c", subcore_axis_name="s",
                                  num_cores=NC, num_subcores=NS)
    half = jax.ShapeDtypeStruct((NC, NR, DH), jnp.float32)
    return pl.kernel(
        _sc_body,
        out_type=(half, half, half),
        mesh=mesh,
        scratch_types=[
            pltpu.VMEM_SHARED((NRP, DH), jnp.float32),
            pltpu.VMEM((NQ, K, 128), jnp.int32),
            pltpu.VMEM((NQ, K, 128), jnp.int32),
            pltpu.VMEM((NQ, CHUNK), jnp.float32),
            pltpu.VMEM((2, CHUNK, DH), jnp.float32),
        ] + [pltpu.SemaphoreType.DMA] * 8,
        compiler_params=pltpu.CompilerParams(use_tc_tiling_on_sc=False),
    )


def _mean_body(a_ref, b_ref, c_ref, d_ref, o_ref):
    o_ref[0] = (a_ref[0] + b_ref[0] + c_ref[0] + d_ref[0]) * 0.25


def kernel(user_emb, item_emb, edge_index, edge_weight):
    e0 = jnp.concatenate([user_emb, item_emb], axis=0)
    e0_st = jnp.stack([e0[:, :DH], e0[:, DH:]], axis=0)
    pad = E_PAD - N_EDGES
    src = jnp.pad(edge_index[0], (0, pad)).reshape(E_PAD // 128, 128)
    dst = jnp.pad(edge_index[1], (0, pad)).reshape(E_PAD // 128, 128)
    w = jnp.pad(edge_weight, (0, pad))

    o1, o2, o3 = _make_sc_kernel()(e0_st, src, dst, w)

    blk = 800
    zs = pl.pallas_call(
        _mean_body,
        out_shape=jax.ShapeDtypeStruct((NC, NR, DH), jnp.float32),
        grid=(NC, NR // blk),
        in_specs=[pl.BlockSpec((1, blk, DH), lambda i, j: (i, j, 0))] * 4,
        out_specs=pl.BlockSpec((1, blk, DH), lambda i, j: (i, j, 0)),
    )(e0_st, o1, o2, o3)

    z = jnp.concatenate([zs[0], zs[1]], axis=1)
    return z[:N_USERS], z[N_USERS:]
